# baseline (speedup 1.0000x reference)
import numpy as np
import jax
import jax.numpy as jnp
from jax import lax
from jax.experimental import pallas as pl
from jax.experimental.pallas import tpu as pltpu
from jax.experimental.pallas import tpu_sc as plsc
import functools

_NS = 32
_NQ_PER_W = 128
_N = 16384
_B = 4
_S = 1024
_INF = float("inf")
_BIGI = 2 ** 30


def _sq_threshold() -> float:
    c = np.float32(0.04)
    x = np.float32(0.0016)
    while np.sqrt(x) >= c:
        x = np.nextafter(x, np.float32(0), dtype=np.float32)
    while np.sqrt(x) < c:
        x = np.nextafter(x, np.float32(1), dtype=np.float32)
    return float(x)


_T = _sq_threshold()


def _take(x, idx):
    return jnp.take_along_axis(x, idx, axis=0)


def _bf16_round(v):
    u = plsc.bitcast(v, jnp.uint32)
    r = (u + jnp.uint32(0x7FFF) + ((u >> jnp.uint32(16)) & jnp.uint32(1)))
    r = r & jnp.uint32(0xFFFF0000)
    return plsc.bitcast(r, jnp.float32)


def _worker(wid, xT_hbm, qT_hbm, out_hbm, xs, ys, zs, qx, qy, qz, outv,
            cs2, cix, xn2, n, s, nq):
    b = wid // 8
    slot = wid % 8
    xbase = b * (3 * n)
    pltpu.sync_copy(xT_hbm.at[pl.ds(xbase, n)], xs)
    pltpu.sync_copy(xT_hbm.at[pl.ds(xbase + n, n)], ys)
    pltpu.sync_copy(xT_hbm.at[pl.ds(xbase + 2 * n, n)], zs)
    qbase = b * (3 * s) + slot * nq
    pltpu.sync_copy(qT_hbm.at[pl.ds(qbase, nq)], qx.at[pl.ds(0, nq)])
    pltpu.sync_copy(qT_hbm.at[pl.ds(qbase + s, nq)], qy.at[pl.ds(0, nq)])
    pltpu.sync_copy(qT_hbm.at[pl.ds(qbase + 2 * s, nq)], qz.at[pl.ds(0, nq)])

    iota16 = lax.broadcasted_iota(jnp.int32, (16,), 0)
    inf16 = jnp.full((16,), _INF, jnp.float32)
    zero16 = jnp.zeros((16,), jnp.int32)
    t = jnp.float32(_T)

    def prep(c, _):
        off = c * 16
        xv = xs[pl.ds(off, 16)]
        yv = ys[pl.ds(off, 16)]
        zv = zs[pl.ds(off, 16)]
        xn2[pl.ds(off, 16)] = (xv * xv + yv * yv) + zv * zv
        xs[pl.ds(off, 16)] = _bf16_round(xv)
        ys[pl.ds(off, 16)] = _bf16_round(yv)
        zs[pl.ds(off, 16)] = _bf16_round(zv)
        return 0

    lax.fori_loop(0, n // 16, prep, 0)

    def per_query(i, _):
        qxs = jnp.full((16,), qx[pl.ds(i, 16)][0], jnp.float32)
        qys = jnp.full((16,), qy[pl.ds(i, 16)][0], jnp.float32)
        qzs = jnp.full((16,), qz[pl.ds(i, 16)][0], jnp.float32)
        qns = (qxs * qxs + qys * qys) + qzs * qzs
        qbx = _bf16_round(qxs)
        qby = _bf16_round(qys)
        qbz = _bf16_round(qzs)

        def dist_chunk(off):
            xv = xs[pl.ds(off, 16)]
            yv = ys[pl.ds(off, 16)]
            zv = zs[pl.ds(off, 16)]
            xnv = xn2[pl.ds(off, 16)]
            dot = (xv * qbx + yv * qby) + zv * qbz
            d2 = (qns + xnv) - jnp.float32(2.0) * dot
            return jnp.maximum(d2, jnp.float32(0.0))

        def group(g, carry):
            gbest, gbidx, cnt = carry
            base = g * 128
            gm = inf16
            for u in range(8):
                off = base + u * 16
                gm = jnp.minimum(gm, dist_chunk(off))
            anyhit = plsc.all_reduce_population_count(gm < t)[0] > 0

            def slow(c):
                def sbody(u, c2):
                    off = base + u * 16
                    s2 = dist_chunk(off)
                    idxv = jnp.full((16,), off, jnp.int32) + iota16
                    hit = s2 < t
                    plsc.store_compressed(cs2.at[pl.ds(c2, 16)], s2,
                                          mask=hit)
                    plsc.store_compressed(cix.at[pl.ds(c2, 16)], idxv,
                                          mask=hit)
                    pc = plsc.all_reduce_population_count(hit)[0]
                    return c2 + pc

                return lax.fori_loop(0, 8, sbody, c)

            cnt = lax.cond(anyhit, slow, lambda c: c, cnt)

            gbetter = plsc.all_reduce_population_count(
                gm < jnp.full((16,), 0.0, jnp.float32) + gbest)[0] > 0

            def improve(args):
                gb, gi = args
                rm, ri = inf16, zero16
                for u in range(8):
                    off = base + u * 16
                    s2 = dist_chunk(off)
                    idxv = jnp.full((16,), off, jnp.int32) + iota16
                    mlt = s2 < rm
                    ri = jnp.where(mlt, idxv, ri)
                    rm = jnp.where(mlt, s2, rm)
                for sh in (8, 4, 2, 1):
                    perm = jnp.bitwise_xor(iota16, sh)
                    om = _take(rm, perm)
                    oi = _take(ri, perm)
                    less = jnp.logical_or(
                        om < rm, jnp.logical_and(om == rm, oi < ri))
                    rm = jnp.where(less, om, rm)
                    ri = jnp.where(less, oi, ri)
                nb, ni = rm[0], ri[0]
                keep = nb < gb
                return jnp.where(keep, nb, gb), jnp.where(keep, ni, gi)

            gbest, gbidx = lax.cond(gbetter, improve, lambda a: a,
                                    (gbest, gbidx))
            return gbest, gbidx, cnt

        gbest, gbidx, cnt = lax.fori_loop(
            0, n // 128, group,
            (jnp.float32(_INF), jnp.int32(0), jnp.int32(0)))

        bi = jnp.full((16,), 0, jnp.int32) + gbidx
        outv[pl.ds(i * _NS, 16)] = bi
        outv[pl.ds(i * _NS + 16, 16)] = bi

        nblk = (cnt + 15) // 16

        def rank_blk(jb, _2):
            ks = cs2[pl.ds(jb * 16, 16)]
            vs = cix[pl.ds(jb * 16, 16)]
            validj = (jnp.full((16,), jb * 16, jnp.int32) + iota16) < cnt
            ks = jnp.where(validj, ks, _INF)

            def src_blk(kb, rank):
                ks2 = cs2[pl.ds(kb * 16, 16)]
                vs2 = cix[pl.ds(kb * 16, 16)]
                validk = (jnp.full((16,), kb * 16, jnp.int32) + iota16) < cnt
                ks2 = jnp.where(validk, ks2, _INF)

                def lane(l, rank):
                    ls = jnp.full((16,), 0, jnp.int32) + l
                    bk = _take(ks2, ls)
                    bv = _take(vs2, ls)
                    less = jnp.logical_or(
                        bk < ks, jnp.logical_and(bk == ks, bv < vs))
                    return rank + jnp.where(less, 1, 0)

                return lax.fori_loop(0, 16, lane, rank)

            rank = lax.fori_loop(0, nblk, src_blk, zero16)
            okm = jnp.logical_and(validj, rank < _NS)
            rank = jnp.minimum(rank, _NS - 1)
            plsc.store_scatter(outv.at[pl.ds(i * _NS, _NS)], [rank], vs,
                               mask=okm)
            return 0

        lax.fori_loop(0, nblk, rank_blk, 0)
        return 0

    lax.fori_loop(0, nq, per_query, 0)
    pltpu.sync_copy(
        outv, out_hbm.at[pl.ds((b * s + slot * nq) * _NS, nq * _NS)])


def _make_sc_kernel():
    mesh = plsc.VectorSubcoreMesh(core_axis_name="c", subcore_axis_name="s")

    @functools.partial(
        pl.kernel,
        mesh=mesh,
        compiler_params=pltpu.CompilerParams(needs_layout_passes=False),
        out_type=jax.ShapeDtypeStruct((_B * _S * _NS,), jnp.int32),
        scratch_types=[
            pltpu.VMEM((_N,), jnp.float32),
            pltpu.VMEM((_N,), jnp.float32),
            pltpu.VMEM((_N,), jnp.float32),
            pltpu.VMEM((_NQ_PER_W + 16,), jnp.float32),
            pltpu.VMEM((_NQ_PER_W + 16,), jnp.float32),
            pltpu.VMEM((_NQ_PER_W + 16,), jnp.float32),
            pltpu.VMEM((_NQ_PER_W * _NS,), jnp.int32),
            pltpu.VMEM((_N + 16,), jnp.float32),
            pltpu.VMEM((_N + 16,), jnp.int32),
            pltpu.VMEM((_N,), jnp.float32),
        ],
    )
    def ball_query_sc(xT_hbm, qT_hbm, out_hbm, xs, ys, zs, qx, qy, qz, outv,
                      cs2, cix, xn2):
        cid = lax.axis_index("c")
        sid = lax.axis_index("s")
        wid = sid * 2 + cid
        _worker(wid, xT_hbm, qT_hbm, out_hbm, xs, ys, zs, qx, qy, qz, outv,
                cs2, cix, xn2, _N, _S, _NQ_PER_W)

    return ball_query_sc


_SC_KERNEL = _make_sc_kernel()


def kernel(xyz, new_xyz):
    B, N, _ = xyz.shape
    S = new_xyz.shape[1]
    xT = jnp.swapaxes(xyz, 1, 2).reshape(-1)
    qT = jnp.swapaxes(new_xyz, 1, 2).reshape(-1)
    out = _SC_KERNEL(xT, qT)
    return out.reshape(B, S, _NS)


# speedup vs baseline: 1.2552x; 1.2552x over previous
import numpy as np
import jax
import jax.numpy as jnp
from jax import lax
from jax.experimental import pallas as pl
from jax.experimental.pallas import tpu as pltpu
from jax.experimental.pallas import tpu_sc as plsc
import functools

_NS = 32
_NQ_PER_W = 128
_N = 16384
_B = 4
_S = 1024
_INF = float("inf")
_BIGI = 2 ** 30


def _sq_threshold() -> float:
    c = np.float32(0.04)
    x = np.float32(0.0016)
    while np.sqrt(x) >= c:
        x = np.nextafter(x, np.float32(0), dtype=np.float32)
    while np.sqrt(x) < c:
        x = np.nextafter(x, np.float32(1), dtype=np.float32)
    return float(x)


_T = _sq_threshold()


def _take(x, idx):
    return jnp.take_along_axis(x, idx, axis=0)


def _bf16_round(v):
    u = plsc.bitcast(v, jnp.uint32)
    r = (u + jnp.uint32(0x7FFF) + ((u >> jnp.uint32(16)) & jnp.uint32(1)))
    r = r & jnp.uint32(0xFFFF0000)
    return plsc.bitcast(r, jnp.float32)


def _worker(wid, xT_hbm, qT_hbm, out_hbm, xs, ys, zs, qx, qy, qz, outv,
            cs2, cix, xn2, n, s, nq):
    b = wid // 8
    slot = wid % 8
    xbase = b * (3 * n)
    pltpu.sync_copy(xT_hbm.at[pl.ds(xbase, n)], xs)
    pltpu.sync_copy(xT_hbm.at[pl.ds(xbase + n, n)], ys)
    pltpu.sync_copy(xT_hbm.at[pl.ds(xbase + 2 * n, n)], zs)
    qbase = b * (3 * s) + slot * nq
    pltpu.sync_copy(qT_hbm.at[pl.ds(qbase, nq)], qx.at[pl.ds(0, nq)])
    pltpu.sync_copy(qT_hbm.at[pl.ds(qbase + s, nq)], qy.at[pl.ds(0, nq)])
    pltpu.sync_copy(qT_hbm.at[pl.ds(qbase + 2 * s, nq)], qz.at[pl.ds(0, nq)])

    iota16 = lax.broadcasted_iota(jnp.int32, (16,), 0)
    inf16 = jnp.full((16,), _INF, jnp.float32)
    zero16 = jnp.zeros((16,), jnp.int32)
    t = jnp.float32(_T)

    def prep(c, _):
        off = c * 16
        xv = xs[pl.ds(off, 16)]
        yv = ys[pl.ds(off, 16)]
        zv = zs[pl.ds(off, 16)]
        xn2[pl.ds(off, 16)] = (xv * xv + yv * yv) + zv * zv
        xs[pl.ds(off, 16)] = _bf16_round(xv)
        ys[pl.ds(off, 16)] = _bf16_round(yv)
        zs[pl.ds(off, 16)] = _bf16_round(zv)
        return 0

    lax.fori_loop(0, n // 16, prep, 0)

    def per_query(i, _):
        qxs = jnp.full((16,), qx[pl.ds(i, 16)][0], jnp.float32)
        qys = jnp.full((16,), qy[pl.ds(i, 16)][0], jnp.float32)
        qzs = jnp.full((16,), qz[pl.ds(i, 16)][0], jnp.float32)
        qns = (qxs * qxs + qys * qys) + qzs * qzs
        qbx = _bf16_round(qxs)
        qby = _bf16_round(qys)
        qbz = _bf16_round(qzs)

        def dist_chunk(off):
            xv = xs[pl.ds(off, 16)]
            yv = ys[pl.ds(off, 16)]
            zv = zs[pl.ds(off, 16)]
            xnv = xn2[pl.ds(off, 16)]
            dot = (xv * qbx + yv * qby) + zv * qbz
            d2 = (qns + xnv) - jnp.float32(2.0) * dot
            return jnp.maximum(d2, jnp.float32(0.0))

        def group(g, carry):
            runmin, runidx, cnt = carry
            base = g * 128
            gm = inf16
            for u in range(8):
                off = base + u * 16
                s2 = dist_chunk(off)
                idxv = jnp.full((16,), off, jnp.int32) + iota16
                mlt = s2 < runmin
                runidx = jnp.where(mlt, idxv, runidx)
                runmin = jnp.where(mlt, s2, runmin)
                gm = jnp.minimum(gm, s2)
            anyhit = plsc.all_reduce_population_count(gm < t)[0] > 0

            def slow(c):
                def sbody(u, c2):
                    off = base + u * 16
                    s2 = dist_chunk(off)
                    idxv = jnp.full((16,), off, jnp.int32) + iota16
                    hit = s2 < t
                    plsc.store_compressed(cs2.at[pl.ds(c2, 16)], s2,
                                          mask=hit)
                    plsc.store_compressed(cix.at[pl.ds(c2, 16)], idxv,
                                          mask=hit)
                    pc = plsc.all_reduce_population_count(hit)[0]
                    return c2 + pc

                return lax.fori_loop(0, 8, sbody, c)

            cnt = lax.cond(anyhit, slow, lambda c: c, cnt)
            return runmin, runidx, cnt

        runmin, runidx, cnt = lax.fori_loop(
            0, n // 128, group, (inf16, zero16, jnp.int32(0)))

        bm, bi = runmin, runidx
        for sh in (8, 4, 2, 1):
            perm = jnp.bitwise_xor(iota16, sh)
            om = _take(bm, perm)
            oi = _take(bi, perm)
            less = jnp.logical_or(
                om < bm, jnp.logical_and(om == bm, oi < bi))
            bm = jnp.where(less, om, bm)
            bi = jnp.where(less, oi, bi)

        outv[pl.ds(i * _NS, 16)] = bi
        outv[pl.ds(i * _NS + 16, 16)] = bi

        nblk = (cnt + 15) // 16

        def rank_blk(jb, _2):
            ks = cs2[pl.ds(jb * 16, 16)]
            vs = cix[pl.ds(jb * 16, 16)]
            validj = (jnp.full((16,), jb * 16, jnp.int32) + iota16) < cnt
            ks = jnp.where(validj, ks, _INF)

            def src_blk(kb, rank):
                ks2 = cs2[pl.ds(kb * 16, 16)]
                vs2 = cix[pl.ds(kb * 16, 16)]
                validk = (jnp.full((16,), kb * 16, jnp.int32) + iota16) < cnt
                ks2 = jnp.where(validk, ks2, _INF)

                def lane(l, rank):
                    ls = jnp.full((16,), 0, jnp.int32) + l
                    bk = _take(ks2, ls)
                    bv = _take(vs2, ls)
                    less = jnp.logical_or(
                        bk < ks, jnp.logical_and(bk == ks, bv < vs))
                    return rank + jnp.where(less, 1, 0)

                return lax.fori_loop(0, 16, lane, rank)

            rank = lax.fori_loop(0, nblk, src_blk, zero16)
            okm = jnp.logical_and(validj, rank < _NS)
            rank = jnp.minimum(rank, _NS - 1)
            plsc.store_scatter(outv.at[pl.ds(i * _NS, _NS)], [rank], vs,
                               mask=okm)
            return 0

        lax.fori_loop(0, nblk, rank_blk, 0)
        return 0

    lax.fori_loop(0, nq, per_query, 0)
    pltpu.sync_copy(
        outv, out_hbm.at[pl.ds((b * s + slot * nq) * _NS, nq * _NS)])


def _make_sc_kernel():
    mesh = plsc.VectorSubcoreMesh(core_axis_name="c", subcore_axis_name="s")

    @functools.partial(
        pl.kernel,
        mesh=mesh,
        compiler_params=pltpu.CompilerParams(needs_layout_passes=False),
        out_type=jax.ShapeDtypeStruct((_B * _S * _NS,), jnp.int32),
        scratch_types=[
            pltpu.VMEM((_N,), jnp.float32),
            pltpu.VMEM((_N,), jnp.float32),
            pltpu.VMEM((_N,), jnp.float32),
            pltpu.VMEM((_NQ_PER_W + 16,), jnp.float32),
            pltpu.VMEM((_NQ_PER_W + 16,), jnp.float32),
            pltpu.VMEM((_NQ_PER_W + 16,), jnp.float32),
            pltpu.VMEM((_NQ_PER_W * _NS,), jnp.int32),
            pltpu.VMEM((_N + 16,), jnp.float32),
            pltpu.VMEM((_N + 16,), jnp.int32),
            pltpu.VMEM((_N,), jnp.float32),
        ],
    )
    def ball_query_sc(xT_hbm, qT_hbm, out_hbm, xs, ys, zs, qx, qy, qz, outv,
                      cs2, cix, xn2):
        cid = lax.axis_index("c")
        sid = lax.axis_index("s")
        wid = sid * 2 + cid
        _worker(wid, xT_hbm, qT_hbm, out_hbm, xs, ys, zs, qx, qy, qz, outv,
                cs2, cix, xn2, _N, _S, _NQ_PER_W)

    return ball_query_sc


_SC_KERNEL = _make_sc_kernel()


def kernel(xyz, new_xyz):
    B, N, _ = xyz.shape
    S = new_xyz.shape[1]
    xT = jnp.swapaxes(xyz, 1, 2).reshape(-1)
    qT = jnp.swapaxes(new_xyz, 1, 2).reshape(-1)
    out = _SC_KERNEL(xT, qT)
    return out.reshape(B, S, _NS)
